# baseline (speedup 1.0000x reference)
import functools
import jax
import jax.numpy as jnp
from jax import lax
from jax.experimental import pallas as pl
from jax.experimental.pallas import tpu as pltpu
from jax.experimental.pallas import tpu_sc as plsc

_N = 10000
_E = 320000
_D = 128
_NC = 2
_NS = 16
_NW = _NC * _NS
_CH = 80
_NCHUNK = 125
_EPT = _NCHUNK * _CH
_EPAD = _NW * _EPT
_NPAD = 10240
_RPT = _NPAD // _NS


def _seg_sum_body(y_hbm, src_hbm, dst_hbm, zer_hbm, out_hbm,
                  acc, srcb0, dstb0, srcb1, dstb1, gbufa, gbufb,
                  gsema, gsemb, isem0, isem1):
    c = lax.axis_index("c")
    s = lax.axis_index("s")
    pltpu.sync_copy(zer_hbm, acc.at[pl.ds(s * _RPT, _RPT)])
    plsc.subcore_barrier()

    cbase = (c * _NS + s) * _EPT

    def istart(g, sb, db, isem):
        off = cbase + g * _CH
        pltpu.async_copy(src_hbm.at[pl.ds(off, _CH)], sb, isem)
        pltpu.async_copy(dst_hbm.at[pl.ds(off, _CH)], db, isem)

    def iwait(g, sb, db, isem):
        off = cbase + g * _CH
        pltpu.make_async_copy(src_hbm.at[pl.ds(off, _CH)], sb, isem).wait()
        pltpu.make_async_copy(dst_hbm.at[pl.ds(off, _CH)], db, isem).wait()

    istart(0, srcb0, dstb0, isem0)
    istart(1, srcb1, dstb1, isem1)
    iwait(0, srcb0, dstb0, isem0)
    pltpu.async_copy(y_hbm.at[srcb0], gbufa, gsema)

    nk = _NCHUNK // 2

    def body(k, carry):
        g0 = 2 * k
        iwait(g0 + 1, srcb1, dstb1, isem1)
        pltpu.make_async_copy(y_hbm.at[srcb0], gbufa, gsema).wait()
        pltpu.async_copy(y_hbm.at[srcb1], gbufb, gsemb)
        pltpu.sync_copy(gbufa, acc.at[dstb0], add=True)
        istart(g0 + 2, srcb0, dstb0, isem0)

        iwait(g0 + 2, srcb0, dstb0, isem0)
        pltpu.make_async_copy(y_hbm.at[srcb1], gbufb, gsemb).wait()
        pltpu.async_copy(y_hbm.at[srcb0], gbufa, gsema)
        pltpu.sync_copy(gbufb, acc.at[dstb1], add=True)

        @pl.when(k < nk - 1)
        def _():
            istart(g0 + 3, srcb1, dstb1, isem1)

        return carry

    lax.fori_loop(0, nk, body, 0)
    pltpu.make_async_copy(y_hbm.at[srcb0], gbufa, gsema).wait()
    pltpu.sync_copy(gbufa, acc.at[dstb0], add=True)
    plsc.subcore_barrier()
    pltpu.sync_copy(acc.at[pl.ds(s * _RPT, _RPT)],
                    out_hbm.at[c, pl.ds(s * _RPT, _RPT)])


_seg_sum = functools.partial(
    pl.kernel,
    out_type=jax.ShapeDtypeStruct((_NC, _NPAD, _D), jnp.float32),
    mesh=plsc.VectorSubcoreMesh(core_axis_name="c", subcore_axis_name="s"),
    scratch_types=[
        pltpu.VMEM_SHARED((_NPAD, _D), jnp.float32),
        pltpu.VMEM((_CH,), jnp.int32),
        pltpu.VMEM((_CH,), jnp.int32),
        pltpu.VMEM((_CH,), jnp.int32),
        pltpu.VMEM((_CH,), jnp.int32),
        pltpu.VMEM((_CH, _D), jnp.float32),
        pltpu.VMEM((_CH, _D), jnp.float32),
        pltpu.SemaphoreType.DMA,
        pltpu.SemaphoreType.DMA,
        pltpu.SemaphoreType.DMA,
        pltpu.SemaphoreType.DMA,
    ],
)(_seg_sum_body)


_BLK = 1000


def _dense_body(final, p_ref, x_ref, wr_ref, br_ref, wt_ref, g_ref, be_ref,
                wl_ref, bl_ref, o_ref):
    agg = p_ref[0] + p_ref[1]
    h = (jnp.dot(agg, wr_ref[...], preferred_element_type=jnp.float32)
         + jnp.dot(x_ref[...], wt_ref[...], preferred_element_type=jnp.float32)
         + br_ref[...])
    m = jnp.mean(h, axis=-1, keepdims=True)
    v = jnp.mean((h - m) * (h - m), axis=-1, keepdims=True)
    ln = (h - m) * lax.rsqrt(v + 1e-5) * g_ref[...] + be_ref[...]
    if final:
        o_ref[...] = (jnp.dot(ln, wl_ref[...],
                              preferred_element_type=jnp.float32)
                      + bl_ref[...])
    else:
        o_ref[...] = ln


def _dense(p, x, w_rel, b_rel, w_root, g, be, w_lin, b_lin, final):
    vec = pl.BlockSpec((1, _D), lambda i: (0, 0))
    mat = pl.BlockSpec((_D, _D), lambda i: (0, 0))
    return pl.pallas_call(
        functools.partial(_dense_body, final),
        grid=(_N // _BLK,),
        in_specs=[
            pl.BlockSpec((2, _BLK, _D), lambda i: (0, i, 0)),
            pl.BlockSpec((_BLK, _D), lambda i: (i, 0)),
            mat, vec, mat, vec, vec, mat, vec,
        ],
        out_specs=pl.BlockSpec((_BLK, _D), lambda i: (i, 0)),
        out_shape=jax.ShapeDtypeStruct((_N, _D), jnp.float32),
    )(p, x, w_rel, b_rel.reshape(1, _D), w_root, g.reshape(1, _D),
      be.reshape(1, _D), w_lin, b_lin.reshape(1, _D))


def kernel(x, edge_index, batch,
           W1_rel, b1_rel, W1_root, g1, be1,
           W2_rel, b2_rel, W2_root, g2, be2,
           W3_rel, b3_rel, W3_root, g3, be3,
           Wlin, blin):
    del batch
    src = edge_index[0]
    dst = edge_index[1]
    zer = jnp.zeros((_RPT, _D), jnp.float32)

    p = _seg_sum(x, src, dst, zer)
    h = _dense(p, x, W1_rel, b1_rel, W1_root, g1, be1, Wlin, blin, False)
    p = _seg_sum(h, src, dst, zer)
    h = _dense(p, h, W2_rel, b2_rel, W2_root, g2, be2, Wlin, blin, False)
    p = _seg_sum(h, src, dst, zer)
    out = _dense(p, h, W3_rel, b3_rel, W3_root, g3, be3, Wlin, blin, True)
    return out
